# baseline (speedup 1.0000x reference)
import jax
import jax.numpy as jnp
from jax import lax
from jax.experimental import pallas as pl
from jax.experimental.pallas import tpu as pltpu
from jax.experimental.pallas import tpu_sc as plsc

_N = 1000
_F = 26
_NCOLS = _F * _N
_NW = 32
_BAND = 8
_LPB = _BAND * _F
_CW = 12800
_TW = 384
_EDGE = 2 * _CW + _TW


def _make_sc_kernel(b_per_w):
    assert b_per_w % _BAND == 0
    nbands = b_per_w // _BAND
    mesh = plsc.VectorSubcoreMesh(core_axis_name="c", subcore_axis_name="s")

    chunks = [(0, _CW), (_CW, _CW), (2 * _CW, _TW)]

    def body(cols_hbm, out_hbm, rowv, colv, buf, sem):
        wid = lax.axis_index("s") * 2 + lax.axis_index("c")
        base = wid * b_per_w
        pltpu.sync_copy(cols_hbm.at[pl.ds(0, _LPB)], rowv)
        pltpu.sync_copy(
            cols_hbm.at[pl.ds(_LPB + base * _F, b_per_w * _F)], colv)

        ones = jnp.full((16,), 1.0, jnp.float32)
        zeros = jnp.zeros((16,), jnp.float32)

        for r in range(_BAND):
            @pl.loop(0, _CW // 16)
            def _(i, r=r):
                buf[r, pl.ds(i * 16, 16)] = zeros

        def scat(bnd, c0, w, val):
            for g in range(_LPB // 16):
                cv = colv[pl.ds(bnd * _LPB + g * 16, 16)]
                rv = rowv[pl.ds(g * 16, 16)]
                m = (cv >= c0) & (cv < c0 + w)
                plsc.store_scatter(buf, [rv, cv - c0], val, mask=m)

        @pl.loop(0, nbands)
        def _(bnd):
            r0 = base + bnd * _BAND
            for c0, w in chunks:
                scat(bnd, c0, w, ones)
                pltpu.async_copy(
                    buf.at[pl.ds(0, _BAND), pl.ds(0, w)],
                    out_hbm.at[pl.ds(r0, _BAND), pl.ds(c0, w)], sem)
                pltpu.make_async_copy(
                    buf.at[pl.ds(0, _BAND), pl.ds(0, w)],
                    out_hbm.at[pl.ds(r0, _BAND), pl.ds(c0, w)], sem).wait()
                scat(bnd, c0, w, zeros)

    return pl.kernel(
        body,
        out_type=jax.ShapeDtypeStruct((b_per_w * _NW, _NCOLS), jnp.float32),
        mesh=mesh,
        scratch_types=[
            pltpu.VMEM((_LPB,), jnp.int32),
            pltpu.VMEM((b_per_w * _F,), jnp.int32),
            pltpu.VMEM((_BAND, _CW), jnp.float32),
            pltpu.SemaphoreType.DMA,
        ],
        compiler_params=pltpu.CompilerParams(
            needs_layout_passes=False, use_tc_tiling_on_sc=True),
    )


def kernel(x):
    b, f = x.shape
    assert f == _F

    cols = x + (_N * jnp.arange(f, dtype=x.dtype))[None, :]

    bp = -(-b // (_NW * _BAND)) * (_NW * _BAND)
    if bp != b:
        cols = jnp.pad(cols, ((0, bp - b), (0, 0)),
                       constant_values=4 * _NCOLS)

    header = (jnp.arange(_LPB, dtype=jnp.int32) // _F) % _BAND
    table = jnp.concatenate([header, cols.reshape(-1)])
    out = _make_sc_kernel(bp // _NW)(table)

    strip = (cols[:, _F - 1:] ==
             _EDGE + jnp.arange(_NCOLS - _EDGE, dtype=jnp.int32)[None, :]
             ).astype(jnp.float32)
    out = jax.lax.dynamic_update_slice(out, strip, (0, _EDGE))
    return out[:b]


# speedup vs baseline: 2.7290x; 2.7290x over previous
import jax
import jax.numpy as jnp
from jax import lax
from jax.experimental import pallas as pl
from jax.experimental.pallas import tpu as pltpu
from jax.experimental.pallas import tpu_sc as plsc

_N = 1000
_F = 26
_NCOLS = _F * _N
_NW = 32
_SW = 128


def _make_sc_kernel(bp):
    assert bp % (_NW * _SW) == 0
    nspw = bp // (_NW * _SW)

    mesh = plsc.VectorSubcoreMesh(core_axis_name="c", subcore_axis_name="s")

    def body(tab_hbm, out_hbm, lanehdr, xbuf, buf, sem):
        wid = lax.axis_index("s") * 2 + lax.axis_index("c")
        pltpu.sync_copy(tab_hbm.at[pl.ds(0, _SW)], lanehdr)

        ones = jnp.full((16,), 1.0, jnp.float32)
        zeros = jnp.zeros((16,), jnp.float32)

        @pl.loop(0, _N)
        def _(i):
            for j in range(_SW // 16):
                buf[i, pl.ds(j * 16, 16)] = zeros

        def scat(val):
            for g in range(_SW // 16):
                xv = xbuf[pl.ds(g * 16, 16)]
                cv = lanehdr[pl.ds(g * 16, 16)]
                plsc.store_scatter(buf, [xv, cv], val, mask=xv < _N)

        for s in range(nspw):
            stripe = wid * nspw + s
            for f in range(_F):
                pltpu.sync_copy(
                    tab_hbm.at[pl.ds(_SW + (stripe * _F + f) * _SW, _SW)],
                    xbuf)
                scat(ones)
                pltpu.async_copy(
                    buf,
                    out_hbm.at[pl.ds(f * _N, _N),
                               pl.ds(stripe * _SW, _SW)], sem)
                pltpu.make_async_copy(
                    buf,
                    out_hbm.at[pl.ds(f * _N, _N),
                               pl.ds(stripe * _SW, _SW)], sem).wait()
                scat(zeros)

    return pl.kernel(
        body,
        out_type=jax.ShapeDtypeStruct((_NCOLS, bp), jnp.float32),
        mesh=mesh,
        scratch_types=[
            pltpu.VMEM((_SW,), jnp.int32),
            pltpu.VMEM((_SW,), jnp.int32),
            pltpu.VMEM((_N, _SW), jnp.float32),
            pltpu.SemaphoreType.DMA,
        ],
        compiler_params=pltpu.CompilerParams(
            needs_layout_passes=False, use_tc_tiling_on_sc=True),
    )


def kernel(x):
    b, f = x.shape
    assert f == _F

    bp = -(-b // (_NW * _SW)) * (_NW * _SW)
    xp = x
    if bp != b:
        xp = jnp.pad(x, ((0, bp - b), (0, 0)), constant_values=_N)

    header = jnp.arange(_SW, dtype=jnp.int32)
    xt = xp.reshape(bp // _SW, _SW, _F).transpose(0, 2, 1).reshape(-1)
    table = jnp.concatenate([header, xt])

    out_t = _make_sc_kernel(bp)(table)
    return out_t.T[:b]
